# baseline (speedup 1.0000x reference)
import jax
import jax.numpy as jnp
from jax.experimental import pallas as pl
from jax.experimental.pallas import tpu as pltpu

D_MODEL_ = 1024
D_OUT_ = 1024
E_ = 8
EG_ = 4
NG_ = E_ // EG_
TM_ = 128


def _moe_body(x_ref, wg_ref, bg_ref, we_ref, be_ref, out_ref, c_ref, acc_ref):
    g = pl.program_id(0)
    t = pl.program_id(1)
    x = x_ref[...]
    sl = pl.ds(t * TM_, TM_)

    @pl.when(g == 0)
    def _gating():
        logits = (
            jnp.dot(x, wg_ref[...], preferred_element_type=jnp.float32)
            + bg_ref[...]
        )
        m1 = jnp.max(logits, axis=-1, keepdims=True)
        oh1 = logits == m1
        l2 = jnp.where(oh1, -jnp.inf, logits)
        m2 = jnp.max(l2, axis=-1, keepdims=True)
        oh2 = l2 == m2
        w1 = jax.nn.sigmoid(m1 - m2)
        w2 = 1.0 - w1
        c = w1 * oh1.astype(jnp.float32) + w2 * oh2.astype(jnp.float32)
        for gg in range(NG_):
            c_ref[gg, sl, :] = c[:, gg * EG_ : (gg + 1) * EG_]
        acc_ref[sl, :] = jnp.dot(
            c, be_ref[...], preferred_element_type=jnp.float32
        )

    cg = c_ref[g, sl, :]
    acc = acc_ref[sl, :]
    for j in range(EG_):
        y = jnp.dot(x, we_ref[j], preferred_element_type=jnp.float32)
        acc = acc + cg[:, j : j + 1] * y

    @pl.when(g < NG_ - 1)
    def _stash():
        acc_ref[sl, :] = acc

    out_ref[...] = acc


def kernel(x, W_e, b_e, W_g, b_g):
    B, S, D = x.shape
    T = B * S
    xf = x.reshape(T, D)
    out = pl.pallas_call(
        _moe_body,
        grid=(NG_, T // TM_),
        in_specs=[
            pl.BlockSpec((TM_, D), lambda g, t: (t, 0)),
            pl.BlockSpec((D, E_), lambda g, t: (0, 0)),
            pl.BlockSpec((1, E_), lambda g, t: (0, 0)),
            pl.BlockSpec((EG_, D, D_OUT_), lambda g, t: (g, 0, 0)),
            pl.BlockSpec((E_, D_OUT_), lambda g, t: (0, 0)),
        ],
        out_specs=pl.BlockSpec((TM_, D_OUT_), lambda g, t: (t, 0)),
        out_shape=jax.ShapeDtypeStruct((T, D_OUT_), jnp.float32),
        scratch_shapes=[
            pltpu.VMEM((NG_, T, EG_), jnp.float32),
            pltpu.VMEM((T, D_OUT_), jnp.float32),
        ],
    )(xf, W_g, b_g.reshape(1, E_), W_e, b_e)
    return out.reshape(B, S, D_OUT_)


# speedup vs baseline: 1.4648x; 1.4648x over previous
import jax
import jax.numpy as jnp
from jax.experimental import pallas as pl
from jax.experimental.pallas import tpu as pltpu

D_MODEL_ = 1024
D_OUT_ = 1024
E_ = 8
TM_ = 1024


def _moe_body(x_ref, wg_ref, bg_ref, we_ref, be_ref, out_ref):
    x = x_ref[...]
    logits = (
        jnp.dot(x, wg_ref[...], preferred_element_type=jnp.float32)
        + bg_ref[...]
    )
    m1 = jnp.max(logits, axis=-1, keepdims=True)
    oh1 = logits == m1
    l2 = jnp.where(oh1, -jnp.inf, logits)
    m2 = jnp.max(l2, axis=-1, keepdims=True)
    oh2 = l2 == m2
    w1 = jax.nn.sigmoid(m1 - m2)
    w2 = 1.0 - w1
    c = w1 * oh1.astype(jnp.float32) + w2 * oh2.astype(jnp.float32)
    acc = jnp.dot(c, be_ref[...], preferred_element_type=jnp.float32)
    for e in range(E_):
        y = jnp.dot(x, we_ref[e], preferred_element_type=jnp.float32)
        acc = acc + c[:, e : e + 1] * y
    out_ref[...] = acc


def kernel(x, W_e, b_e, W_g, b_g):
    B, S, D = x.shape
    T = B * S
    xf = x.reshape(T, D)
    out = pl.pallas_call(
        _moe_body,
        grid=(T // TM_,),
        in_specs=[
            pl.BlockSpec((TM_, D), lambda i: (i, 0)),
            pl.BlockSpec((D, E_), lambda i: (0, 0)),
            pl.BlockSpec((1, E_), lambda i: (0, 0)),
            pl.BlockSpec((E_, D, D_OUT_), lambda i: (0, 0, 0)),
            pl.BlockSpec((E_, D_OUT_), lambda i: (0, 0)),
        ],
        out_specs=pl.BlockSpec((TM_, D_OUT_), lambda i: (i, 0)),
        out_shape=jax.ShapeDtypeStruct((T, D_OUT_), jnp.float32),
        compiler_params=pltpu.CompilerParams(
            vmem_limit_bytes=100 * 1024 * 1024
        ),
    )(xf, W_g, b_g.reshape(1, E_), W_e, b_e)
    return out.reshape(B, S, D_OUT_)
